# baseline (speedup 1.0000x reference)
import functools

import jax
import jax.numpy as jnp
from jax.experimental import pallas as pl
from jax.experimental.pallas import tpu as pltpu


def _body(x_ref, t_ref, o_ref, *, scale):
    o_ref[...] = x_ref[...] + t_ref[...] * scale


def kernel(inputs, lookup_table):
    B, T, U = inputs.shape
    scale = float(U) ** 0.5

    BB = 64
    grid = (B // BB,)
    out = pl.pallas_call(
        functools.partial(_body, scale=scale),
        grid=grid,
        in_specs=[
            pl.BlockSpec((BB, T, U), lambda i: (i, 0, 0)),
            pl.BlockSpec((T, U), lambda i: (0, 0)),
        ],
        out_specs=pl.BlockSpec((BB, T, U), lambda i: (i, 0, 0)),
        out_shape=jax.ShapeDtypeStruct((B, T, U), jnp.float32),
        compiler_params=pltpu.CompilerParams(
            dimension_semantics=("arbitrary",),
        ),
    )(inputs, lookup_table)
    return out


# speedup vs baseline: 6.3548x; 6.3548x over previous
import functools

import jax
import jax.numpy as jnp
from jax.experimental import pallas as pl
from jax.experimental.pallas import tpu as pltpu


def _body(x_ref, t_ref, o_ref, *, scale):
    t = t_ref[...] * scale
    o_ref[...] = x_ref[...] + t[:, :, None]


def kernel(inputs, lookup_table):
    B, T, U = inputs.shape
    scale = float(U) ** 0.5

    x = jnp.transpose(inputs, (1, 2, 0))

    BT = 8
    grid = (T // BT,)
    out = pl.pallas_call(
        functools.partial(_body, scale=scale),
        grid=grid,
        in_specs=[
            pl.BlockSpec((BT, U, B), lambda i: (i, 0, 0)),
            pl.BlockSpec((BT, U), lambda i: (i, 0)),
        ],
        out_specs=pl.BlockSpec((BT, U, B), lambda i: (i, 0, 0)),
        out_shape=jax.ShapeDtypeStruct((T, U, B), jnp.float32),
        compiler_params=pltpu.CompilerParams(
            dimension_semantics=("arbitrary",),
        ),
    )(x, lookup_table)
    return jnp.transpose(out, (2, 0, 1))
